# baseline (speedup 1.0000x reference)
import functools

import jax
import jax.numpy as jnp
from jax import lax
from jax.experimental import pallas as pl
from jax.experimental.pallas import tpu as pltpu
from jax.experimental.pallas import tpu_sc as plsc

NUM_EMB = 1_000_000
D = 64
NB = 16384
NSEQ = 20
C = 128
NITEM = NSEQ * (NB // C)
NC, NS = 2, 16
NW = NC * NS
IPW = NITEM // NW
VBLK = NB // C

NT = (NUM_EMB + 127) // 128
TPW = (NT + NW - 1) // NW
TP_ROWS = NT * D

_mesh = plsc.VectorSubcoreMesh(core_axis_name="c", subcore_axis_name="s")

_params = pltpu.CompilerParams(
    use_tc_tiling_on_sc=True, needs_layout_passes=False)


@functools.partial(
    pl.kernel,
    out_type=jax.ShapeDtypeStruct((TP_ROWS, 128), jnp.float32),
    mesh=_mesh,
    scratch_types=[
        pltpu.VMEM((D, C), jnp.float32),
        pltpu.VMEM((D, C), jnp.float32),
    ],
    compiler_params=_params,
)
def _pack_table(tt_hbm, tp_hbm, stage_v, outp_v):
    wid = lax.axis_index("s") * NC + lax.axis_index("c")
    lane = lax.iota(jnp.int32, 16)
    rowsel = [jnp.bitwise_and(lane + 16 * q, 63) for q in range(8)]
    step = [lax.shift_right_logical(lane + 16 * q, 6) for q in range(8)]

    def body(k, carry):
        t = wid + NW * k

        @pl.when(t < NT)
        def _():
            pltpu.sync_copy(tt_hbm.at[:, pl.ds(t * C, C)], stage_v)
            for j in range(D):
                for q in range(8):
                    v = plsc.load_gather(stage_v, [rowsel[q], step[q] + 2 * j])
                    outp_v[j, pl.ds(16 * q, 16)] = v
            pltpu.sync_copy(outp_v, tp_hbm.at[pl.ds(t * D, D)])

        return carry

    lax.fori_loop(0, TPW, body, 0)


@functools.partial(
    pl.kernel,
    out_type=jax.ShapeDtypeStruct((NSEQ, D, NB), jnp.float32),
    mesh=_mesh,
    scratch_types=[
        pltpu.VMEM((IPW, C), jnp.int32),
        pltpu.VMEM((C,), jnp.int32),
        pltpu.VMEM((C, 128), jnp.float32),
        pltpu.VMEM((D, C), jnp.float32),
        pltpu.SemaphoreType.DMA,
    ],
    compiler_params=_params,
)
def _emb_lookup(tok_hbm, tp_hbm, out_hbm, tok_v, idx2_v, rows_v, outt_v, gsem):
    wid = lax.axis_index("s") * NC + lax.axis_index("c")
    base_item = wid * IPW
    pltpu.sync_copy(tok_hbm.at[pl.ds(base_item, IPW)], tok_v)

    lane = lax.iota(jnp.int32, 16)

    def body(i, carry):
        for q in range(8):
            t = tok_v[i, pl.ds(16 * q, 16)]
            idx2_v[pl.ds(16 * q, 16)] = lax.shift_right_logical(t, 1)
        pltpu.async_copy(tp_hbm.at[idx2_v], rows_v, gsem).wait()

        for q in range(8):
            rowsel = lane + (16 * q)
            t = tok_v[i, pl.ds(16 * q, 16)]
            paroff = lax.shift_left(jnp.bitwise_and(t, 1), 6)
            for d in range(D):
                v = plsc.load_gather(rows_v, [rowsel, paroff + d])
                outt_v[d, pl.ds(16 * q, 16)] = v

        item = base_item + i
        s = item // VBLK
        vb = item % VBLK
        pltpu.sync_copy(outt_v, out_hbm.at[s, :, pl.ds(vb * C, C)])
        return carry

    lax.fori_loop(0, IPW, body, 0)


def kernel(token_ids, embeddings):
    tp = _pack_table(embeddings.T)
    tok = token_ids.T.reshape(NITEM, C).astype(jnp.int32)
    out3 = _emb_lookup(tok, tp)
    return out3.transpose(2, 0, 1)


# speedup vs baseline: 1.0080x; 1.0080x over previous
import functools

import jax
import jax.numpy as jnp
from jax import lax
from jax.experimental import pallas as pl
from jax.experimental.pallas import tpu as pltpu
from jax.experimental.pallas import tpu_sc as plsc

NUM_EMB = 1_000_000
D = 64
NB = 16384
NSEQ = 20
C = 128
NITEM = NSEQ * (NB // C)
NC, NS = 2, 16
NW = NC * NS
IPW = NITEM // NW
VBLK = NB // C

NT = (NUM_EMB + 127) // 128
TPW = (NT + NW - 1) // NW
TP_ROWS = NT * D

_mesh = plsc.VectorSubcoreMesh(core_axis_name="c", subcore_axis_name="s")

_params = pltpu.CompilerParams(
    use_tc_tiling_on_sc=True, needs_layout_passes=False)


@functools.partial(
    pl.kernel,
    out_type=jax.ShapeDtypeStruct((TP_ROWS, 128), jnp.float32),
    mesh=_mesh,
    scratch_types=[
        pltpu.VMEM((D, C + 1), jnp.float32),
        pltpu.VMEM((D, C), jnp.float32),
    ],
    compiler_params=_params,
)
def _pack_table(tt_hbm, tp_hbm, stage_v, outp_v):
    wid = lax.axis_index("s") * NC + lax.axis_index("c")
    lane = lax.iota(jnp.int32, 16)
    rowsel = [jnp.bitwise_and(lane + 16 * q, 63) for q in range(8)]
    step = [lax.shift_right_logical(lane + 16 * q, 6) for q in range(8)]

    def body(k, carry):
        t = wid + NW * k

        @pl.when(t < NT)
        def _():
            pltpu.sync_copy(tt_hbm.at[:, pl.ds(t * C, C)],
                            stage_v.at[:, pl.ds(0, C)])

            def tbody(j, c2):
                for q in range(8):
                    v = plsc.load_gather(stage_v, [rowsel[q], step[q] + 2 * j])
                    outp_v[j, pl.ds(16 * q, 16)] = v
                return c2

            lax.fori_loop(0, D, tbody, 0)
            pltpu.sync_copy(outp_v, tp_hbm.at[pl.ds(t * D, D)])

        return carry

    lax.fori_loop(0, TPW, body, 0)


@functools.partial(
    pl.kernel,
    out_type=jax.ShapeDtypeStruct((NSEQ, D, NB), jnp.float32),
    mesh=_mesh,
    scratch_types=[
        pltpu.VMEM((IPW, C), jnp.int32),
        pltpu.VMEM((C,), jnp.int32),
        pltpu.VMEM((C, 129), jnp.float32),
        pltpu.VMEM((D, C), jnp.float32),
        pltpu.SemaphoreType.DMA,
    ],
    compiler_params=_params,
)
def _emb_lookup(tok_hbm, tp_hbm, out_hbm, tok_v, idx2_v, rows_v, outt_v, gsem):
    wid = lax.axis_index("s") * NC + lax.axis_index("c")
    base_item = wid * IPW
    pltpu.sync_copy(tok_hbm.at[pl.ds(base_item, IPW)], tok_v)

    lane = lax.iota(jnp.int32, 16)

    def body(i, carry):
        for q in range(8):
            t = tok_v[i, pl.ds(16 * q, 16)]
            idx2_v[pl.ds(16 * q, 16)] = lax.shift_right_logical(t, 1)
        pltpu.async_copy(tp_hbm.at[idx2_v], rows_v.at[:, pl.ds(0, 128)],
                         gsem).wait()

        rowsels = [lane + 16 * q for q in range(8)]
        paroffs = []
        for q in range(8):
            t = tok_v[i, pl.ds(16 * q, 16)]
            paroffs.append(lax.shift_left(jnp.bitwise_and(t, 1), 6))

        def tbody(d, c2):
            for q in range(8):
                v = plsc.load_gather(rows_v, [rowsels[q], paroffs[q] + d])
                outt_v[d, pl.ds(16 * q, 16)] = v
            return c2

        lax.fori_loop(0, D, tbody, 0)

        item = base_item + i
        s = item // VBLK
        vb = item % VBLK
        pltpu.sync_copy(outt_v, out_hbm.at[s, :, pl.ds(vb * C, C)])
        return carry

    lax.fori_loop(0, IPW, body, 0)


def kernel(token_ids, embeddings):
    tp = _pack_table(embeddings.T)
    tok = token_ids.T.reshape(NITEM, C).astype(jnp.int32)
    out3 = _emb_lookup(tok, tp)
    return out3.transpose(2, 0, 1)


# speedup vs baseline: 1.6113x; 1.5985x over previous
import functools

import jax
import jax.numpy as jnp
from jax import lax
from jax.experimental import pallas as pl
from jax.experimental.pallas import tpu as pltpu
from jax.experimental.pallas import tpu_sc as plsc

NUM_EMB = 1_000_000
D = 64
NB = 16384
NSEQ = 20
C = 128
NITEM = NSEQ * (NB // C)
NC, NS = 2, 16
NW = NC * NS
IPW = NITEM // NW
VBLK = NB // C

NT = (NUM_EMB + 127) // 128
TPW = (NT + NW - 1) // NW
TP_ROWS = NT * D

_mesh = plsc.VectorSubcoreMesh(core_axis_name="c", subcore_axis_name="s")

_params = pltpu.CompilerParams(
    use_tc_tiling_on_sc=True, needs_layout_passes=False)


@functools.partial(
    pl.kernel,
    out_type=jax.ShapeDtypeStruct((TP_ROWS, 128), jnp.float32),
    mesh=_mesh,
    scratch_types=[
        pltpu.VMEM((D, C + 1), jnp.float32),
        pltpu.VMEM((D, C), jnp.float32),
    ],
    compiler_params=_params,
)
def _pack_table(tt_hbm, tp_hbm, stage_v, outp_v):
    wid = lax.axis_index("s") * NC + lax.axis_index("c")
    lane = lax.iota(jnp.int32, 16)
    rowsel = [jnp.bitwise_and(lane + 16 * q, 63) for q in range(8)]
    step = [lax.shift_right_logical(lane + 16 * q, 6) for q in range(8)]

    def body(k, carry):
        t = wid + NW * k

        @pl.when(t < NT)
        def _():
            pltpu.sync_copy(tt_hbm.at[:, pl.ds(t * C, C)],
                            stage_v.at[:, pl.ds(0, C)])

            @plsc.parallel_loop(0, D, unroll=4)
            def tbody(j):
                for q in range(8):
                    v = plsc.load_gather(stage_v, [rowsel[q], step[q] + 2 * j])
                    outp_v[j, pl.ds(16 * q, 16)] = v
            pltpu.sync_copy(outp_v, tp_hbm.at[pl.ds(t * D, D)])

        return carry

    lax.fori_loop(0, TPW, body, 0)


@functools.partial(
    pl.kernel,
    out_type=jax.ShapeDtypeStruct((NSEQ, D, NB), jnp.float32),
    mesh=_mesh,
    scratch_types=[
        pltpu.VMEM((IPW, C), jnp.int32),
        pltpu.VMEM((C,), jnp.int32),
        pltpu.VMEM((C, 129), jnp.float32),
        pltpu.VMEM((D, C), jnp.float32),
        pltpu.SemaphoreType.DMA,
    ],
    compiler_params=_params,
)
def _emb_lookup(tok_hbm, tp_hbm, out_hbm, tok_v, idx2_v, rows_v, outt_v, gsem):
    wid = lax.axis_index("s") * NC + lax.axis_index("c")
    base_item = wid * IPW
    pltpu.sync_copy(tok_hbm.at[pl.ds(base_item, IPW)], tok_v)

    lane = lax.iota(jnp.int32, 16)

    def body(i, carry):
        for q in range(8):
            t = tok_v[i, pl.ds(16 * q, 16)]
            idx2_v[pl.ds(16 * q, 16)] = lax.shift_right_logical(t, 1)
        pltpu.async_copy(tp_hbm.at[idx2_v], rows_v.at[:, pl.ds(0, 128)],
                         gsem).wait()

        rowsels = [lane + 16 * q for q in range(8)]
        paroffs = []
        for q in range(8):
            t = tok_v[i, pl.ds(16 * q, 16)]
            paroffs.append(lax.shift_left(jnp.bitwise_and(t, 1), 6))

        @plsc.parallel_loop(0, D, unroll=4)
        def tbody(d):
            for q in range(8):
                v = plsc.load_gather(rows_v, [rowsels[q], paroffs[q] + d])
                outt_v[d, pl.ds(16 * q, 16)] = v

        item = base_item + i
        s = item // VBLK
        vb = item % VBLK
        pltpu.sync_copy(outt_v, out_hbm.at[s, :, pl.ds(vb * C, C)])
        return carry

    lax.fori_loop(0, IPW, body, 0)


def kernel(token_ids, embeddings):
    tp = _pack_table(embeddings.T)
    tok = token_ids.T.reshape(NITEM, C).astype(jnp.int32)
    out3 = _emb_lookup(tok, tp)
    return out3.transpose(2, 0, 1)


# speedup vs baseline: 2.5859x; 1.6048x over previous
import functools

import jax
import jax.numpy as jnp
from jax import lax
from jax.experimental import pallas as pl
from jax.experimental.pallas import tpu as pltpu
from jax.experimental.pallas import tpu_sc as plsc

NUM_EMB = 1_000_000
D = 64
NB = 16384
NSEQ = 20
C = 128
NITEM = NSEQ * (NB // C)
NC, NS = 2, 16
NW = NC * NS
IPW = NITEM // NW
VBLK = NB // C

NT = (NUM_EMB + 127) // 128
TPW = (NT + NW - 1) // NW
TCW = 512
HALF = 500224
NTC2 = HALF // TCW
TP_ROWS = HALF

NBUF = 4
GA = 3

_mesh = plsc.VectorSubcoreMesh(core_axis_name="c", subcore_axis_name="s")

_params = pltpu.CompilerParams(
    use_tc_tiling_on_sc=True, needs_layout_passes=False)


def _tc_pack_body(ta_ref, tb_ref, tp_ref):
    tp_ref[:, 0:64] = ta_ref[...].T
    tp_ref[:, 64:128] = tb_ref[...].T


_pack_table = pl.pallas_call(
    _tc_pack_body,
    grid=(NTC2,),
    in_specs=[
        pl.BlockSpec((D, TCW), lambda g: (0, g)),
        pl.BlockSpec((D, TCW), lambda g: (0, g + NTC2)),
    ],
    out_specs=pl.BlockSpec((TCW, 128), lambda g: (g, 0)),
    out_shape=jax.ShapeDtypeStruct((TP_ROWS, 128), jnp.float32),
)


@functools.partial(
    pl.kernel,
    out_type=jax.ShapeDtypeStruct((NSEQ, D, NB), jnp.float32),
    mesh=_mesh,
    scratch_types=[
        pltpu.VMEM((IPW, C), jnp.int32),
        pltpu.VMEM((C,), jnp.int32),
        pltpu.VMEM((C,), jnp.int32),
        pltpu.VMEM((C, 129), jnp.float32),
        pltpu.VMEM((C, 129), jnp.float32),
        pltpu.VMEM((D, C), jnp.float32),
        pltpu.VMEM((D, C), jnp.float32),
        pltpu.SemaphoreType.DMA,
        pltpu.SemaphoreType.DMA,
    ],
    compiler_params=_params,
)
def _emb_lookup(tok_hbm, tp_hbm, out_hbm, tok_v, i0, i1,
                r0, r1, o0, o1, gsem, ssem):
    idx2s = [i0, i1]
    rows = [r0, r1]
    outts = [o0, o1]
    wid = lax.axis_index("s") * NC + lax.axis_index("c")
    base_item = wid * IPW
    pltpu.sync_copy(tok_hbm.at[pl.ds(base_item, IPW)], tok_v)

    lane = lax.iota(jnp.int32, 16)
    rowsels = [lane + 16 * q for q in range(8)]

    def start_gather(i, slot):
        icl = jnp.minimum(i, IPW - 1)
        for q in range(8):
            t = tok_v[icl, pl.ds(16 * q, 16)]
            idx2s[slot][pl.ds(16 * q, 16)] = jnp.where(
                t >= HALF, t - HALF, t)
        pltpu.async_copy(tp_hbm.at[idx2s[slot]],
                         rows[slot].at[:, pl.ds(0, 128)], gsem)

    def wait_gather(slot):
        pltpu.make_async_copy(tp_hbm.at[idx2s[slot]],
                              rows[slot].at[:, pl.ds(0, 128)], gsem).wait()

    def start_store(i, slot):
        item = base_item + i
        s = item // VBLK
        vb = item % VBLK
        pltpu.async_copy(outts[slot],
                         out_hbm.at[s, :, pl.ds(vb * C, C)], ssem)

    def wait_store(slot):
        pltpu.make_async_copy(outts[slot],
                              out_hbm.at[0, :, pl.ds(0, C)], ssem).wait()

    start_gather(0, 0)

    def body(g, carry):
        for b in range(2):
            i = g * 2 + b
            wait_gather(b)
            start_gather(i + 1, (b + 1) % 2)

            @pl.when(i >= 2)
            def _():
                wait_store(b % 2)

            paroffs = []
            for q in range(8):
                t = tok_v[i, pl.ds(16 * q, 16)]
                paroffs.append(jnp.where(t >= HALF, 64, 0))

            @plsc.parallel_loop(0, D, unroll=4)
            def tbody(d):
                for q in range(8):
                    v = plsc.load_gather(rows[b],
                                         [rowsels[q], paroffs[q] + d])
                    outts[b % 2][d, pl.ds(16 * q, 16)] = v

            start_store(i, b % 2)
        return carry

    lax.fori_loop(0, IPW // 2, body, 0)

    wait_gather(IPW % 2)
    wait_store(0)
    wait_store(1)


def kernel(token_ids, embeddings):
    tt = embeddings.T
    tp = _pack_table(tt, tt)
    tok = token_ids.T.reshape(NITEM, C).astype(jnp.int32)
    out3 = _emb_lookup(tok, tp)
    return out3.transpose(2, 0, 1)
